# baseline (speedup 1.0000x reference)
import functools

import jax
import jax.numpy as jnp
from jax.experimental import pallas as pl


def _fused_kernel(x_ref, w1_ref, b1_ref, w2_ref, b2_ref, o_ref):
    x = x_ref[...]
    scores = jnp.dot(x, w1_ref[...], preferred_element_type=jnp.float32)
    scores = scores + b1_ref[...]
    R, H = scores.shape
    col = jax.lax.broadcasted_iota(jnp.int32, (R, H), 1)
    m1 = jnp.max(scores, axis=1, keepdims=True)
    i1 = jnp.min(jnp.where(scores == m1, col, H), axis=1, keepdims=True)
    mask1 = col == i1
    rest = jnp.where(mask1, -jnp.inf, scores)
    m2 = jnp.max(rest, axis=1, keepdims=True)
    i2 = jnp.min(jnp.where(rest == m2, col, H), axis=1, keepdims=True)
    sel = jnp.where(mask1 | (col == i2), scores, 0.0)
    o_ref[...] = jnp.dot(sel, w2_ref[...], preferred_element_type=jnp.float32) + b2_ref[...]


@functools.partial(jax.jit, static_argnames=("block_rows",))
def _run(events2d, W1, b1, W2, b2, block_rows):
    n_rows, d = events2d.shape
    h = W1.shape[1]
    grid = (n_rows // block_rows,)
    out = pl.pallas_call(
        _fused_kernel,
        grid=grid,
        in_specs=[
            pl.BlockSpec((block_rows, d), lambda i: (i, 0)),
            pl.BlockSpec((d, h), lambda i: (0, 0)),
            pl.BlockSpec((1, h), lambda i: (0, 0)),
            pl.BlockSpec((h, 1), lambda i: (0, 0)),
            pl.BlockSpec((1, 1), lambda i: (0, 0)),
        ],
        out_specs=pl.BlockSpec((block_rows, 1), lambda i: (i, 0)),
        out_shape=jax.ShapeDtypeStruct((n_rows, 1), jnp.float32),
    )(events2d, W1, b1.reshape(1, h), W2, b2.reshape(1, 1))
    return out


def kernel(events, W1, b1, W2, b2):
    B, S, D = events.shape
    n_rows = B * S
    block_rows = 1024 if n_rows % 1024 == 0 else 8
    out = _run(events.reshape(n_rows, D), W1, b1, W2, b2, block_rows)
    return out.reshape(B, S, 1)


# speedup vs baseline: 1.0928x; 1.0928x over previous
import functools

import jax
import jax.numpy as jnp
from jax.experimental import pallas as pl


def _fused_kernel(x_ref, w1_ref, b1_ref, w2_ref, b2_ref, o_ref):
    x = x_ref[...]
    scores = jnp.dot(x, w1_ref[...], preferred_element_type=jnp.float32)
    scores = scores + b1_ref[...]
    R, H = scores.shape
    col = jax.lax.broadcasted_iota(jnp.int32, (R, H), 1)
    m1 = jnp.max(scores, axis=1, keepdims=True)
    i1 = jnp.min(jnp.where(scores == m1, col, H), axis=1, keepdims=True)
    mask1 = col == i1
    rest = jnp.where(mask1, -jnp.inf, scores)
    m2 = jnp.max(rest, axis=1, keepdims=True)
    i2 = jnp.min(jnp.where(rest == m2, col, H), axis=1, keepdims=True)
    sel = jnp.where(mask1 | (col == i2), scores, 0.0)
    o_ref[...] = jnp.dot(sel, w2_ref[...], preferred_element_type=jnp.float32) + b2_ref[...]


@functools.partial(jax.jit, static_argnames=("block_rows",))
def _run(events2d, W1, b1, W2, b2, block_rows):
    n_rows, d = events2d.shape
    h = W1.shape[1]
    grid = (n_rows // block_rows,)
    out = pl.pallas_call(
        _fused_kernel,
        grid=grid,
        in_specs=[
            pl.BlockSpec((block_rows, d), lambda i: (i, 0)),
            pl.BlockSpec((d, h), lambda i: (0, 0)),
            pl.BlockSpec((1, h), lambda i: (0, 0)),
            pl.BlockSpec((h, 1), lambda i: (0, 0)),
            pl.BlockSpec((1, 1), lambda i: (0, 0)),
        ],
        out_specs=pl.BlockSpec((block_rows, 1), lambda i: (i, 0)),
        out_shape=jax.ShapeDtypeStruct((n_rows, 1), jnp.float32),
    )(events2d, W1, b1.reshape(1, h), W2, b2.reshape(1, 1))
    return out


def kernel(events, W1, b1, W2, b2):
    B, S, D = events.shape
    n_rows = B * S
    block_rows = 2048 if n_rows % 2048 == 0 else 8
    out = _run(events.reshape(n_rows, D), W1, b1, W2, b2, block_rows)
    return out.reshape(B, S, 1)
